# Initial kernel scaffold; baseline (speedup 1.0000x reference)
import jax
import jax.numpy as jnp
from jax.experimental import pallas as pl


def kernel(x, edge_index, W1, b1, W2, b2, Wf, bf):
    raise NotImplementedError("write your pallas kernel here")


# speedup vs baseline: 16.5737x; 16.5737x over previous
import functools

import jax
import jax.numpy as jnp
from jax import lax
from jax.experimental import pallas as pl
from jax.experimental.pallas import tpu as pltpu
from jax.experimental.pallas import tpu_sc as plsc

N = 10000
E = 320000
D = 128
H = 64
C = 40

NC = 2
NS = 16
CH = 128
NCHUNK = (E + NC * NS * CH - 1) // (NC * NS * CH)
EPAD = NC * NS * NCHUNK * CH
NPAD = 10240
RPT = NPAD // NS
PAD_IDX = NPAD - 1

BM = 1024
BMF = 1000


def _mesh():
    return plsc.VectorSubcoreMesh(core_axis_name="c", subcore_axis_name="s")


_SC_PARAMS = pltpu.CompilerParams(use_tc_tiling_on_sc=False)


def _deg_body(dstw, ones_hbm, zeros_hbm, out, dst_v, ones_v, acc):
    c = lax.axis_index("c")
    s = lax.axis_index("s")
    pltpu.sync_copy(zeros_hbm.at[pl.ds(s * RPT, RPT)], acc.at[pl.ds(s * RPT, RPT)])
    pltpu.sync_copy(dstw.at[c, s], dst_v)
    pltpu.sync_copy(ones_hbm, ones_v)
    plsc.subcore_barrier()

    def chunk(i, carry):
        pltpu.sync_copy(ones_v, acc.at[dst_v.at[i]], add=True)
        return carry

    lax.fori_loop(0, NCHUNK, chunk, 0)
    plsc.subcore_barrier()
    pltpu.sync_copy(acc.at[pl.ds(s * RPT, RPT)], out.at[c, pl.ds(s * RPT, RPT)])


def _deg_call(dstw, ones16, zeros16):
    return pl.kernel(
        _deg_body,
        out_type=jax.ShapeDtypeStruct((NC, NPAD, 16), jnp.float32),
        mesh=_mesh(),
        scratch_types=[
            pltpu.VMEM((NCHUNK, CH), jnp.int32),
            pltpu.VMEM((CH, 16), jnp.float32),
            pltpu.VMEM_SHARED((NPAD, 16), jnp.float32),
        ],
        compiler_params=_SC_PARAMS,
    )(dstw, ones16, zeros16)


def _agg_body(u_hbm, srcw, dstw, zeros_hbm, out, src_v, dst_v, rows_v, acc, sem):
    c = lax.axis_index("c")
    s = lax.axis_index("s")
    pltpu.sync_copy(zeros_hbm.at[pl.ds(s * RPT, RPT)], acc.at[pl.ds(s * RPT, RPT)])
    pltpu.sync_copy(srcw.at[c, s], src_v)
    pltpu.sync_copy(dstw.at[c, s], dst_v)
    plsc.subcore_barrier()

    def chunk(i, carry):
        pltpu.async_copy(u_hbm.at[src_v.at[i]], rows_v, sem).wait()
        pltpu.sync_copy(rows_v, acc.at[dst_v.at[i]], add=True)
        return carry

    lax.fori_loop(0, NCHUNK, chunk, 0)
    plsc.subcore_barrier()
    pltpu.sync_copy(acc.at[pl.ds(s * RPT, RPT)], out.at[c, pl.ds(s * RPT, RPT)])


def _agg_call(u, srcw, dstw, zeros64):
    return pl.kernel(
        _agg_body,
        out_type=jax.ShapeDtypeStruct((NC, NPAD, H), jnp.float32),
        mesh=_mesh(),
        scratch_types=[
            pltpu.VMEM((NCHUNK, CH), jnp.int32),
            pltpu.VMEM((NCHUNK, CH), jnp.int32),
            pltpu.VMEM((CH, H), jnp.float32),
            pltpu.VMEM_SHARED((NPAD, H), jnp.float32),
            pltpu.SemaphoreType.DMA,
        ],
        compiler_params=_SC_PARAMS,
    )(u, srcw, dstw, zeros64)


def _dinv_block(deg_ref, base_rows):
    deg = deg_ref[0][:, 0:1] + deg_ref[1][:, 0:1] + 1.0
    rows = base_rows + lax.broadcasted_iota(jnp.int32, deg.shape, 0)
    return jnp.where(rows < N, lax.rsqrt(deg), 0.0)


def _mm_scale_body(deg_ref, x_ref, w_ref, o_ref):
    i = pl.program_id(0)
    dinv = _dinv_block(deg_ref, i * BM)
    g = jnp.dot(x_ref[...], w_ref[...], preferred_element_type=jnp.float32)
    o_ref[...] = g * dinv


def _mm_scale(degp, x_pad, W1):
    return pl.pallas_call(
        _mm_scale_body,
        grid=(NPAD // BM,),
        in_specs=[
            pl.BlockSpec((NC, BM, 16), lambda i: (0, i, 0)),
            pl.BlockSpec((BM, D), lambda i: (i, 0)),
            pl.BlockSpec((D, H), lambda i: (0, 0)),
        ],
        out_specs=pl.BlockSpec((BM, H), lambda i: (i, 0)),
        out_shape=jax.ShapeDtypeStruct((NPAD, H), jnp.float32),
    )(degp, x_pad, W1)


def _comb_mm_body(deg_ref, sp_ref, u_ref, b_ref, w_ref, o_ref):
    i = pl.program_id(0)
    dinv = _dinv_block(deg_ref, i * BM)
    t = sp_ref[0] + sp_ref[1] + u_ref[...]
    h = jnp.maximum(dinv * t + b_ref[...], 0.0)
    o_ref[...] = dinv * jnp.dot(h, w_ref[...], preferred_element_type=jnp.float32)


def _comb_mm(degp, sp, u, b_row, W):
    return pl.pallas_call(
        _comb_mm_body,
        grid=(NPAD // BM,),
        in_specs=[
            pl.BlockSpec((NC, BM, 16), lambda i: (0, i, 0)),
            pl.BlockSpec((NC, BM, H), lambda i: (0, i, 0)),
            pl.BlockSpec((BM, H), lambda i: (i, 0)),
            pl.BlockSpec((1, H), lambda i: (0, 0)),
            pl.BlockSpec((H, H), lambda i: (0, 0)),
        ],
        out_specs=pl.BlockSpec((BM, H), lambda i: (i, 0)),
        out_shape=jax.ShapeDtypeStruct((NPAD, H), jnp.float32),
    )(degp, sp, u, b_row, W)


def _comb_body(deg_ref, sp_ref, u_ref, b_ref, o_ref):
    i = pl.program_id(0)
    dinv = _dinv_block(deg_ref, i * BM)
    t = sp_ref[0] + sp_ref[1] + u_ref[...]
    o_ref[...] = dinv * jnp.maximum(dinv * t + b_ref[...], 0.0)


def _comb(degp, sp, u, b_row):
    return pl.pallas_call(
        _comb_body,
        grid=(NPAD // BM,),
        in_specs=[
            pl.BlockSpec((NC, BM, 16), lambda i: (0, i, 0)),
            pl.BlockSpec((NC, BM, H), lambda i: (0, i, 0)),
            pl.BlockSpec((BM, H), lambda i: (i, 0)),
            pl.BlockSpec((1, H), lambda i: (0, 0)),
        ],
        out_specs=pl.BlockSpec((BM, H), lambda i: (i, 0)),
        out_shape=jax.ShapeDtypeStruct((NPAD, H), jnp.float32),
    )(degp, sp, u, b_row)


def _final_body(deg_ref, sp_ref, u_ref, wf_ref, bf_ref, o_ref):
    deg = deg_ref[0][:, 0:1] + deg_ref[1][:, 0:1] + 1.0
    dinv = lax.rsqrt(deg)
    t = dinv * (sp_ref[0] + sp_ref[1] + u_ref[...])
    z = jnp.dot(t, wf_ref[...], preferred_element_type=jnp.float32) + bf_ref[...]
    m = jnp.max(z, axis=1, keepdims=True)
    lse = m + jnp.log(jnp.sum(jnp.exp(z - m), axis=1, keepdims=True))
    o_ref[...] = z - lse


def _final(degp, sp, u, Wf, bf_row):
    return pl.pallas_call(
        _final_body,
        grid=(N // BMF,),
        in_specs=[
            pl.BlockSpec((NC, BMF, 16), lambda i: (0, i, 0)),
            pl.BlockSpec((NC, BMF, H), lambda i: (0, i, 0)),
            pl.BlockSpec((BMF, H), lambda i: (i, 0)),
            pl.BlockSpec((H, C), lambda i: (0, 0)),
            pl.BlockSpec((1, C), lambda i: (0, 0)),
        ],
        out_specs=pl.BlockSpec((BMF, C), lambda i: (i, 0)),
        out_shape=jax.ShapeDtypeStruct((N, C), jnp.float32),
    )(degp, sp, u, Wf, bf_row)


def kernel(x, edge_index, W1, b1, W2, b2, Wf, bf):
    ei = edge_index.astype(jnp.int32)
    pad = jnp.full((EPAD - E,), PAD_IDX, jnp.int32)
    srcw = jnp.concatenate([ei[0], pad]).reshape(NC, NS, NCHUNK, CH)
    dstw = jnp.concatenate([ei[1], pad]).reshape(NC, NS, NCHUNK, CH)

    x_pad = jnp.pad(x, ((0, NPAD - N), (0, 0)))
    zeros16 = jnp.zeros((NPAD, 16), jnp.float32)
    zeros64 = jnp.zeros((NPAD, H), jnp.float32)
    ones16 = jnp.ones((CH, 16), jnp.float32)
    b1r = b1.reshape(1, H)
    b2r = b2.reshape(1, H)
    bfr = bf.reshape(1, C)

    degp = _deg_call(dstw, ones16, zeros16)
    u1 = _mm_scale(degp, x_pad, W1)
    s1 = _agg_call(u1, srcw, dstw, zeros64)
    u2 = _comb_mm(degp, s1, u1, b1r, W2)
    s2 = _agg_call(u2, srcw, dstw, zeros64)
    u3 = _comb(degp, s2, u2, b2r)
    s3 = _agg_call(u3, srcw, dstw, zeros64)
    return _final(degp, s3, u3, Wf, bfr)
